# baseline (speedup 1.0000x reference)
import jax
import jax.numpy as jnp
from jax import lax
from jax.experimental import pallas as pl
from jax.experimental.pallas import tpu as pltpu
from jax.experimental.pallas import tpu_sc as plsc

N_NODES = 10000
E_EDGES = 320000
D = 128

CHUNK = 80
NUM_CORES = 2
NUM_SUBCORES = 16
NUM_WORKERS = NUM_CORES * NUM_SUBCORES
EDGES_PER_WORKER = E_EDGES // NUM_WORKERS
CHUNKS_PER_WORKER = EDGES_PER_WORKER // CHUNK
N_PAD = 10240
ROWS_PER_TILE = N_PAD // NUM_SUBCORES

ROW_BLK = 400
NUM_ROW_BLKS = N_NODES // ROW_BLK


def _linear_body(x_ref, wt_ref, b_ref, o_ref):
    o_ref[...] = (
        jnp.dot(x_ref[...], wt_ref[...], preferred_element_type=jnp.float32)
        + b_ref[...]
    )


def _linear(x, Wt, b2):
    return pl.pallas_call(
        _linear_body,
        grid=(NUM_ROW_BLKS,),
        in_specs=[
            pl.BlockSpec((ROW_BLK, D), lambda i: (i, 0)),
            pl.BlockSpec((D, D), lambda i: (0, 0)),
            pl.BlockSpec((1, D), lambda i: (0, 0)),
        ],
        out_specs=pl.BlockSpec((ROW_BLK, D), lambda i: (i, 0)),
        out_shape=jax.ShapeDtypeStruct((N_NODES, D), jnp.float32),
    )(x, Wt, b2)


def _combine_body(p0_ref, p1_ref, o_ref):
    o_ref[...] = p0_ref[0] + p1_ref[0]


def _combine(partials):
    return pl.pallas_call(
        _combine_body,
        grid=(NUM_ROW_BLKS,),
        in_specs=[
            pl.BlockSpec((1, ROW_BLK, D), lambda i: (0, i, 0)),
            pl.BlockSpec((1, ROW_BLK, D), lambda i: (1, i, 0)),
        ],
        out_specs=pl.BlockSpec((ROW_BLK, D), lambda i: (i, 0)),
        out_shape=jax.ShapeDtypeStruct((N_NODES, D), jnp.float32),
    )(partials, partials)


NBUF = 2


def _sc_body(h_hbm, src_hbm, dst_hbm, w_hbm, out_hbm,
             src_all, rows0, rows1, dst0, dst1, w0, w1, acc,
             sg0, sg1, ss0, ss1, sd0, sd1, sw0, sw1):
    rows = [rows0, rows1]
    dst = [dst0, dst1]
    wbuf = [w0, w1]
    sem_g = [sg0, sg1]
    sem_s = [ss0, ss1]
    sem_d = [sd0, sd1]
    sem_w = [sw0, sw1]

    cid = lax.axis_index("c")
    sid = lax.axis_index("s")
    wid = cid * NUM_SUBCORES + sid
    cpw = CHUNKS_PER_WORKER

    ebase = wid * EDGES_PER_WORKER

    zeros16 = jnp.zeros((16,), jnp.float32)

    def zero_row(i, _):
        for j in range(D // 16):
            rows0[i, pl.ds(j * 16, 16)] = zeros16
        return 0

    lax.fori_loop(0, CHUNK, zero_row, 0)

    row0 = sid * ROWS_PER_TILE
    for k in range(ROWS_PER_TILE // CHUNK):
        pltpu.sync_copy(rows0, acc.at[pl.ds(row0 + k * CHUNK, CHUNK)])
    plsc.subcore_barrier()

    pltpu.sync_copy(src_hbm.at[pl.ds(ebase, EDGES_PER_WORKER)], src_all)

    def gather_start(k, b):
        pltpu.async_copy(h_hbm.at[src_all.at[pl.ds(k * CHUNK, CHUNK)]],
                         rows[b], sem_g[b])

    def gather_wait(k, b):
        pltpu.make_async_copy(h_hbm.at[src_all.at[pl.ds(0, CHUNK)]],
                              rows[b], sem_g[b]).wait()

    def dw_start(k, b):
        pltpu.async_copy(dst_hbm.at[pl.ds(ebase + k * CHUNK, CHUNK)],
                         dst[b], sem_d[b])
        pltpu.async_copy(w_hbm.at[pl.ds(ebase + k * CHUNK, CHUNK)],
                         wbuf[b], sem_w[b])

    def dw_wait(k, b):
        pltpu.make_async_copy(dst_hbm.at[pl.ds(ebase + k * CHUNK, CHUNK)],
                              dst[b], sem_d[b]).wait()
        pltpu.make_async_copy(w_hbm.at[pl.ds(ebase + k * CHUNK, CHUNK)],
                              wbuf[b], sem_w[b]).wait()

    def scatter_start(k, b):
        pltpu.async_copy(rows[b], acc.at[dst[b]], sem_s[b], add=True)

    def scatter_wait(k, b):
        pltpu.make_async_copy(rows[b], acc.at[dst[b]], sem_s[b]).wait()

    gather_start(0, 0)
    dw_start(0, 0)

    def group_body(g, _):
        for b in range(NBUF):
            k = g * NBUF + b
            pb = (b + 1) % NBUF

            @pl.when(k >= 1)
            def _():
                scatter_wait(k - 1, pb)

            @pl.when(k + 1 < cpw)
            def _():
                gather_start(k + 1, pb)
                dw_start(k + 1, pb)

            gather_wait(k, b)
            dw_wait(k, b)

            def mul_group(gg, _):
                w16 = wbuf[b][pl.ds(gg * 16, 16)]
                for l in range(16):
                    wv = jnp.full((16,), w16[l], jnp.float32)
                    i = gg * 16 + l
                    for j in range(D // 16):
                        sl = pl.ds(j * 16, 16)
                        rows[b][i, sl] = rows[b][i, sl] * wv
                return 0

            lax.fori_loop(0, CHUNK // 16, mul_group, 0)
            scatter_start(k, b)
        return 0

    lax.fori_loop(0, cpw // NBUF, group_body, 0)

    kt = cpw - 1
    scatter_wait(kt - 1, 1)
    gather_wait(kt, 0)
    dw_wait(kt, 0)

    def mul_tail(gg, _):
        w16 = wbuf[0][pl.ds(gg * 16, 16)]
        for l in range(16):
            wv = jnp.full((16,), w16[l], jnp.float32)
            i = gg * 16 + l
            for j in range(D // 16):
                sl = pl.ds(j * 16, 16)
                rows[0][i, sl] = rows[0][i, sl] * wv
        return 0

    lax.fori_loop(0, CHUNK // 16, mul_tail, 0)
    scatter_start(kt, 0)
    scatter_wait(kt, 0)
    plsc.subcore_barrier()

    pltpu.sync_copy(acc.at[pl.ds(row0, ROWS_PER_TILE)],
                    out_hbm.at[cid, pl.ds(row0, ROWS_PER_TILE)])


def _scatter_gather(h, src, dst_ix, w):
    mesh = plsc.VectorSubcoreMesh(core_axis_name="c", subcore_axis_name="s")
    run = pl.kernel(
        _sc_body,
        mesh=mesh,
        out_type=jax.ShapeDtypeStruct((NUM_CORES, N_PAD, D), jnp.float32),
        scratch_types=(
            [pltpu.VMEM((EDGES_PER_WORKER,), jnp.int32)]
            + [pltpu.VMEM((CHUNK, D), jnp.float32)] * NBUF
            + [pltpu.VMEM((CHUNK,), jnp.int32)] * NBUF
            + [pltpu.VMEM((CHUNK,), jnp.float32)] * NBUF
            + [pltpu.VMEM_SHARED((N_PAD, D), jnp.float32)]
            + [pltpu.SemaphoreType.DMA] * (4 * NBUF)
        ),
    )
    return run(h, src, dst_ix, w)


@jax.jit
def kernel(x, edge_index, w, W, b):
    h = _linear(x, W.T, b[None, :])
    partials = _scatter_gather(h, edge_index[0], edge_index[1], w)
    return _combine(partials)


# speedup vs baseline: 1.0491x; 1.0491x over previous
import jax
import jax.numpy as jnp
from jax import lax
from jax.experimental import pallas as pl
from jax.experimental.pallas import tpu as pltpu
from jax.experimental.pallas import tpu_sc as plsc

N_NODES = 10000
E_EDGES = 320000
D = 128

CHUNK = 128
NUM_CORES = 2
NUM_SUBCORES = 16
NUM_WORKERS = NUM_CORES * NUM_SUBCORES
NUM_CHUNKS = E_EDGES // CHUNK
CHUNKS_BASE = NUM_CHUNKS // NUM_WORKERS
CHUNKS_REM = NUM_CHUNKS % NUM_WORKERS
SRC_STAGE = CHUNKS_BASE + 1
N_PAD = 10240
ROWS_PER_TILE = N_PAD // NUM_SUBCORES

ROW_BLK = 400
NUM_ROW_BLKS = N_NODES // ROW_BLK


def _linear_body(x_ref, wt_ref, b_ref, o_ref):
    o_ref[...] = (
        jnp.dot(x_ref[...], wt_ref[...], preferred_element_type=jnp.float32)
        + b_ref[...]
    )


def _linear(x, Wt, b2):
    return pl.pallas_call(
        _linear_body,
        grid=(NUM_ROW_BLKS,),
        in_specs=[
            pl.BlockSpec((ROW_BLK, D), lambda i: (i, 0)),
            pl.BlockSpec((D, D), lambda i: (0, 0)),
            pl.BlockSpec((1, D), lambda i: (0, 0)),
        ],
        out_specs=pl.BlockSpec((ROW_BLK, D), lambda i: (i, 0)),
        out_shape=jax.ShapeDtypeStruct((N_NODES, D), jnp.float32),
    )(x, Wt, b2)


def _combine_body(p0_ref, p1_ref, o_ref):
    o_ref[...] = p0_ref[0] + p1_ref[0]


def _combine(partials):
    return pl.pallas_call(
        _combine_body,
        grid=(NUM_ROW_BLKS,),
        in_specs=[
            pl.BlockSpec((1, ROW_BLK, D), lambda i: (0, i, 0)),
            pl.BlockSpec((1, ROW_BLK, D), lambda i: (1, i, 0)),
        ],
        out_specs=pl.BlockSpec((ROW_BLK, D), lambda i: (i, 0)),
        out_shape=jax.ShapeDtypeStruct((N_NODES, D), jnp.float32),
    )(partials, partials)


NBUF = 2


def _sc_body(h_hbm, src_hbm, dst_hbm, w_hbm, out_hbm,
             src_all, rows0, rows1, dst0, dst1, w0, w1, acc,
             sg0, sg1, ss0, ss1, sd0, sd1, sw0, sw1):
    rows = [rows0, rows1]
    dst = [dst0, dst1]
    wbuf = [w0, w1]
    sem_g = [sg0, sg1]
    sem_s = [ss0, ss1]
    sem_d = [sd0, sd1]
    sem_w = [sw0, sw1]

    cid = lax.axis_index("c")
    sid = lax.axis_index("s")
    wid = cid * NUM_SUBCORES + sid
    extra = (wid < CHUNKS_REM).astype(jnp.int32)
    cpw = CHUNKS_BASE + extra
    cbase = CHUNKS_BASE * wid + jnp.minimum(wid, CHUNKS_REM)

    zeros16 = jnp.zeros((16,), jnp.float32)

    def zero_row(i, _):
        for j in range(D // 16):
            rows0[i, pl.ds(j * 16, 16)] = zeros16
        return 0

    lax.fori_loop(0, CHUNK, zero_row, 0)

    row0 = sid * ROWS_PER_TILE
    for k in range(ROWS_PER_TILE // CHUNK):
        pltpu.sync_copy(rows0, acc.at[pl.ds(row0 + k * CHUNK, CHUNK)])
    plsc.subcore_barrier()

    pltpu.sync_copy(src_hbm.at[pl.ds(cbase * CHUNK, SRC_STAGE * CHUNK)],
                    src_all)

    def gather_start(k, b):
        pltpu.async_copy(h_hbm.at[src_all.at[pl.ds(k * CHUNK, CHUNK)]],
                         rows[b], sem_g[b])

    def gather_wait(k, b):
        pltpu.make_async_copy(h_hbm.at[src_all.at[pl.ds(0, CHUNK)]],
                              rows[b], sem_g[b]).wait()

    def dw_start(k, b):
        e0 = (cbase + k) * CHUNK
        pltpu.async_copy(dst_hbm.at[pl.ds(e0, CHUNK)], dst[b], sem_d[b])
        pltpu.async_copy(w_hbm.at[pl.ds(e0, CHUNK)], wbuf[b], sem_w[b])

    def dw_wait(k, b):
        e0 = (cbase + k) * CHUNK
        pltpu.make_async_copy(dst_hbm.at[pl.ds(e0, CHUNK)], dst[b],
                              sem_d[b]).wait()
        pltpu.make_async_copy(w_hbm.at[pl.ds(e0, CHUNK)], wbuf[b],
                              sem_w[b]).wait()

    def scatter_start(k, b):
        pltpu.async_copy(rows[b], acc.at[dst[b]], sem_s[b], add=True)

    def scatter_wait(k, b):
        pltpu.make_async_copy(rows[b], acc.at[dst[b]], sem_s[b]).wait()

    gather_start(0, 0)
    dw_start(0, 0)

    def group_body(g, _):
        for b in range(NBUF):
            k = g * NBUF + b
            pb = (b + 1) % NBUF

            @pl.when(k >= 1)
            def _():
                scatter_wait(k - 1, pb)

            @pl.when(k + 1 < cpw)
            def _():
                gather_start(k + 1, pb)
                dw_start(k + 1, pb)

            gather_wait(k, b)
            dw_wait(k, b)

            def mul_group(gg, _):
                w16 = wbuf[b][pl.ds(gg * 16, 16)]
                for l in range(16):
                    wv = jnp.full((16,), w16[l], jnp.float32)
                    i = gg * 16 + l
                    for j in range(D // 16):
                        sl = pl.ds(j * 16, 16)
                        rows[b][i, sl] = rows[b][i, sl] * wv
                return 0

            lax.fori_loop(0, CHUNK // 16, mul_group, 0)
            scatter_start(k, b)
        return 0

    lax.fori_loop(0, CHUNKS_BASE // NBUF, group_body, 0)

    scatter_wait(CHUNKS_BASE - 1, 1)

    @pl.when(extra == 1)
    def _():
        kt = CHUNKS_BASE
        gather_wait(kt, 0)
        dw_wait(kt, 0)

        def mul_tail(gg, _):
            w16 = wbuf[0][pl.ds(gg * 16, 16)]
            for l in range(16):
                wv = jnp.full((16,), w16[l], jnp.float32)
                i = gg * 16 + l
                for j in range(D // 16):
                    sl = pl.ds(j * 16, 16)
                    rows[0][i, sl] = rows[0][i, sl] * wv
            return 0

        lax.fori_loop(0, CHUNK // 16, mul_tail, 0)
        scatter_start(kt, 0)
        scatter_wait(kt, 0)

    plsc.subcore_barrier()

    pltpu.sync_copy(acc.at[pl.ds(row0, ROWS_PER_TILE)],
                    out_hbm.at[cid, pl.ds(row0, ROWS_PER_TILE)])


def _scatter_gather(h, src_p, dst_ix, w):
    mesh = plsc.VectorSubcoreMesh(core_axis_name="c", subcore_axis_name="s")
    run = pl.kernel(
        _sc_body,
        mesh=mesh,
        out_type=jax.ShapeDtypeStruct((NUM_CORES, N_PAD, D), jnp.float32),
        scratch_types=(
            [pltpu.VMEM((SRC_STAGE * CHUNK,), jnp.int32)]
            + [pltpu.VMEM((CHUNK, D), jnp.float32)] * NBUF
            + [pltpu.VMEM((CHUNK,), jnp.int32)] * NBUF
            + [pltpu.VMEM((CHUNK,), jnp.float32)] * NBUF
            + [pltpu.VMEM_SHARED((N_PAD, D), jnp.float32)]
            + [pltpu.SemaphoreType.DMA] * (4 * NBUF)
        ),
    )
    return run(h, src_p, dst_ix, w)


@jax.jit
def kernel(x, edge_index, w, W, b):
    h = _linear(x, W.T, b[None, :])

    src_p = jnp.concatenate(
        [edge_index[0], jnp.zeros((CHUNK,), jnp.int32)])

    partials = _scatter_gather(h, src_p, edge_index[1], w)
    return _combine(partials)


# speedup vs baseline: 1.0651x; 1.0153x over previous
import jax
import jax.numpy as jnp
from jax import lax
from jax.experimental import pallas as pl
from jax.experimental.pallas import tpu as pltpu
from jax.experimental.pallas import tpu_sc as plsc

N_NODES = 10000
E_EDGES = 320000
D = 128

CHUNK = 128
E_PAD = 327680
NUM_CORES = 2
NUM_SUBCORES = 16
NUM_WORKERS = NUM_CORES * NUM_SUBCORES
CHUNKS_PER_WORKER = (E_PAD // CHUNK) // NUM_WORKERS
N_PAD = 10240
ROWS_PER_TILE = N_PAD // NUM_SUBCORES

ROW_BLK = 400
NUM_ROW_BLKS = N_NODES // ROW_BLK


def _linear_body(x_ref, wt_ref, b_ref, o_ref):
    o_ref[...] = (
        jnp.dot(x_ref[...], wt_ref[...], preferred_element_type=jnp.float32)
        + b_ref[...]
    )


def _linear(x, Wt, b2):
    return pl.pallas_call(
        _linear_body,
        grid=(NUM_ROW_BLKS,),
        in_specs=[
            pl.BlockSpec((ROW_BLK, D), lambda i: (i, 0)),
            pl.BlockSpec((D, D), lambda i: (0, 0)),
            pl.BlockSpec((1, D), lambda i: (0, 0)),
        ],
        out_specs=pl.BlockSpec((ROW_BLK, D), lambda i: (i, 0)),
        out_shape=jax.ShapeDtypeStruct((N_NODES, D), jnp.float32),
    )(x, Wt, b2)


def _combine_body(p0_ref, p1_ref, o_ref):
    o_ref[...] = p0_ref[0] + p1_ref[0]


def _combine(partials):
    return pl.pallas_call(
        _combine_body,
        grid=(NUM_ROW_BLKS,),
        in_specs=[
            pl.BlockSpec((1, ROW_BLK, D), lambda i: (0, i, 0)),
            pl.BlockSpec((1, ROW_BLK, D), lambda i: (1, i, 0)),
        ],
        out_specs=pl.BlockSpec((ROW_BLK, D), lambda i: (i, 0)),
        out_shape=jax.ShapeDtypeStruct((N_NODES, D), jnp.float32),
    )(partials, partials)


NBUF = 2


def _sc_body(h_hbm, src_hbm, dst_hbm, w_hbm, out_hbm,
             src_all, rows0, rows1, dst0, dst1, w0, w1, acc,
             sg0, sg1, ss0, ss1, sd0, sd1, sw0, sw1):
    rows = [rows0, rows1]
    dst = [dst0, dst1]
    wbuf = [w0, w1]
    sem_g = [sg0, sg1]
    sem_s = [ss0, ss1]
    sem_d = [sd0, sd1]
    sem_w = [sw0, sw1]

    cid = lax.axis_index("c")
    sid = lax.axis_index("s")
    wid = cid * NUM_SUBCORES + sid
    cpw = CHUNKS_PER_WORKER

    zeros16 = jnp.zeros((16,), jnp.float32)

    def zero_row(i, _):
        for j in range(D // 16):
            rows0[i, pl.ds(j * 16, 16)] = zeros16
        return 0

    lax.fori_loop(0, CHUNK, zero_row, 0)

    row0 = sid * ROWS_PER_TILE
    for k in range(ROWS_PER_TILE // CHUNK):
        pltpu.sync_copy(rows0, acc.at[pl.ds(row0 + k * CHUNK, CHUNK)])
    plsc.subcore_barrier()

    pltpu.sync_copy(src_hbm.at[pl.ds(wid * cpw, cpw)], src_all)

    def gather_start(k, b):
        pltpu.async_copy(h_hbm.at[src_all.at[k]], rows[b], sem_g[b])

    def gather_wait(k, b):
        pltpu.make_async_copy(h_hbm.at[src_all.at[k]], rows[b],
                              sem_g[b]).wait()

    def dw_start(k, b):
        pltpu.async_copy(dst_hbm.at[pl.ds(wid * cpw + k, 1)], dst[b],
                         sem_d[b])
        pltpu.async_copy(w_hbm.at[wid * cpw + k], wbuf[b], sem_w[b])

    def dw_wait(k, b):
        pltpu.make_async_copy(dst_hbm.at[pl.ds(wid * cpw + k, 1)], dst[b],
                              sem_d[b]).wait()
        pltpu.make_async_copy(w_hbm.at[wid * cpw + k], wbuf[b],
                              sem_w[b]).wait()

    def scatter_start(k, b):
        pltpu.async_copy(rows[b], acc.at[dst[b].at[0]], sem_s[b], add=True)

    def scatter_wait(k, b):
        pltpu.make_async_copy(rows[b], acc.at[dst[b].at[0]],
                              sem_s[b]).wait()

    gather_start(0, 0)
    dw_start(0, 0)

    def group_body(g, _):
        for b in range(NBUF):
            k = g * NBUF + b
            pb = (b + 1) % NBUF

            @pl.when(k >= 1)
            def _():
                scatter_wait(k - 1, pb)

            @pl.when(k + 1 < cpw)
            def _():
                gather_start(k + 1, pb)
                dw_start(k + 1, pb)

            gather_wait(k, b)
            dw_wait(k, b)

            def mul_group(gg, _):
                w16 = wbuf[b][pl.ds(gg * 16, 16)]
                for l in range(16):
                    wv = jnp.full((16,), w16[l], jnp.float32)
                    i = gg * 16 + l
                    for j in range(D // 16):
                        sl = pl.ds(j * 16, 16)
                        rows[b][i, sl] = rows[b][i, sl] * wv
                return 0

            lax.fori_loop(0, CHUNK // 16, mul_group, 0)
            scatter_start(k, b)
        return 0

    lax.fori_loop(0, cpw // NBUF, group_body, 0)

    scatter_wait(cpw - 1, (cpw - 1) % NBUF)
    plsc.subcore_barrier()

    pltpu.sync_copy(acc.at[pl.ds(row0, ROWS_PER_TILE)],
                    out_hbm.at[cid, pl.ds(row0, ROWS_PER_TILE)])


def _scatter_gather(h, src_c, dst_c, w_c):
    mesh = plsc.VectorSubcoreMesh(core_axis_name="c", subcore_axis_name="s")
    run = pl.kernel(
        _sc_body,
        mesh=mesh,
        out_type=jax.ShapeDtypeStruct((NUM_CORES, N_PAD, D), jnp.float32),
        scratch_types=(
            [pltpu.VMEM((CHUNKS_PER_WORKER, CHUNK), jnp.int32)]
            + [pltpu.VMEM((CHUNK, D), jnp.float32)] * NBUF
            + [pltpu.VMEM((1, CHUNK), jnp.int32)] * NBUF
            + [pltpu.VMEM((CHUNK,), jnp.float32)] * NBUF
            + [pltpu.VMEM_SHARED((N_PAD, D), jnp.float32)]
            + [pltpu.SemaphoreType.DMA] * (4 * NBUF)
        ),
    )
    return run(h, src_c, dst_c, w_c)


@jax.jit
def kernel(x, edge_index, w, W, b):
    h = _linear(x, W.T, b[None, :])

    pad = E_PAD - E_EDGES
    pad_idx = (jnp.arange(pad, dtype=jnp.int32) * 37) % N_NODES
    edge_pad = jnp.concatenate(
        [edge_index, jnp.stack([pad_idx, pad_idx])], axis=1)
    w_pad = jnp.concatenate([w, jnp.zeros((pad,), jnp.float32)])

    src_c = edge_pad[0].reshape(E_PAD // CHUNK, CHUNK)
    dst_c = edge_pad[1].reshape(E_PAD // CHUNK, CHUNK)
    w_c = w_pad.reshape(E_PAD // CHUNK, CHUNK)

    partials = _scatter_gather(h, src_c, dst_c, w_c)
    return _combine(partials)


# speedup vs baseline: 1.1679x; 1.0965x over previous
import jax
import jax.numpy as jnp
from jax import lax
from jax.experimental import pallas as pl
from jax.experimental.pallas import tpu as pltpu
from jax.experimental.pallas import tpu_sc as plsc

N_NODES = 10000
E_EDGES = 320000
D = 128

CHUNK = 112
E_PAD = 322560
NUM_CORES = 2
NUM_SUBCORES = 16
NUM_WORKERS = NUM_CORES * NUM_SUBCORES
CHUNKS_PER_WORKER = (E_PAD // CHUNK) // NUM_WORKERS
N_PAD = 10240
ROWS_PER_TILE = N_PAD // NUM_SUBCORES

ROW_BLK = 400
NUM_ROW_BLKS = N_NODES // ROW_BLK


def _linear_body(x_ref, wt_ref, b_ref, o_ref):
    o_ref[...] = (
        jnp.dot(x_ref[...], wt_ref[...], preferred_element_type=jnp.float32)
        + b_ref[...]
    )


def _linear(x, Wt, b2):
    return pl.pallas_call(
        _linear_body,
        grid=(NUM_ROW_BLKS,),
        in_specs=[
            pl.BlockSpec((ROW_BLK, D), lambda i: (i, 0)),
            pl.BlockSpec((D, D), lambda i: (0, 0)),
            pl.BlockSpec((1, D), lambda i: (0, 0)),
        ],
        out_specs=pl.BlockSpec((ROW_BLK, D), lambda i: (i, 0)),
        out_shape=jax.ShapeDtypeStruct((N_NODES, D), jnp.float32),
    )(x, Wt, b2)


def _combine_body(p0_ref, p1_ref, o_ref):
    o_ref[...] = p0_ref[0] + p1_ref[0]


def _combine(partials):
    return pl.pallas_call(
        _combine_body,
        grid=(NUM_ROW_BLKS,),
        in_specs=[
            pl.BlockSpec((1, ROW_BLK, D), lambda i: (0, i, 0)),
            pl.BlockSpec((1, ROW_BLK, D), lambda i: (1, i, 0)),
        ],
        out_specs=pl.BlockSpec((ROW_BLK, D), lambda i: (i, 0)),
        out_shape=jax.ShapeDtypeStruct((N_NODES, D), jnp.float32),
    )(partials, partials)


NBUF = 3


def _sc_body(h_hbm, src_hbm, dst_hbm, w_hbm, out_hbm,
             rows0, rows1, rows2, sidx0, sidx1, sidx2,
             dst0, dst1, dst2, w0, w1, w2, acc,
             sg0, sg1, sg2, ss0, ss1, ss2,
             sd0, sd1, sd2, sw0, sw1, sw2, si0, si1, si2):
    rows = [rows0, rows1, rows2]
    sidx = [sidx0, sidx1, sidx2]
    dst = [dst0, dst1, dst2]
    wbuf = [w0, w1, w2]
    sem_g = [sg0, sg1, sg2]
    sem_s = [ss0, ss1, ss2]
    sem_d = [sd0, sd1, sd2]
    sem_w = [sw0, sw1, sw2]
    sem_i = [si0, si1, si2]

    cid = lax.axis_index("c")
    sid = lax.axis_index("s")
    wid = cid * NUM_SUBCORES + sid
    cpw = CHUNKS_PER_WORKER
    ebase = wid * cpw * CHUNK

    zeros16 = jnp.zeros((16,), jnp.float32)

    def zero_row(i, _):
        for j in range(D // 16):
            rows0[i, pl.ds(j * 16, 16)] = zeros16
        return 0

    lax.fori_loop(0, CHUNK, zero_row, 0)

    row0 = sid * ROWS_PER_TILE
    for k in range(ROWS_PER_TILE // CHUNK):
        pltpu.sync_copy(rows0, acc.at[pl.ds(row0 + k * CHUNK, CHUNK)])
    rem = ROWS_PER_TILE % CHUNK
    pltpu.sync_copy(
        rows0.at[pl.ds(0, rem)],
        acc.at[pl.ds(row0 + (ROWS_PER_TILE // CHUNK) * CHUNK, rem)])
    plsc.subcore_barrier()

    def src_start(k, j):
        pltpu.async_copy(src_hbm.at[pl.ds(ebase + k * CHUNK, CHUNK)],
                         sidx[j], sem_i[j])

    def src_wait(k, j):
        pltpu.make_async_copy(src_hbm.at[pl.ds(ebase + k * CHUNK, CHUNK)],
                              sidx[j], sem_i[j]).wait()

    def gather_start(k, b):
        pltpu.async_copy(h_hbm.at[sidx[b]], rows[b], sem_g[b])

    def gather_wait(k, b):
        pltpu.make_async_copy(h_hbm.at[sidx[b]], rows[b], sem_g[b]).wait()

    def dw_start(k, b):
        pltpu.async_copy(dst_hbm.at[pl.ds(ebase + k * CHUNK, CHUNK)],
                         dst[b], sem_d[b])
        pltpu.async_copy(w_hbm.at[pl.ds(ebase + k * CHUNK, CHUNK)],
                         wbuf[b], sem_w[b])

    def dw_wait(k, b):
        pltpu.make_async_copy(dst_hbm.at[pl.ds(ebase + k * CHUNK, CHUNK)],
                              dst[b], sem_d[b]).wait()
        pltpu.make_async_copy(w_hbm.at[pl.ds(ebase + k * CHUNK, CHUNK)],
                              wbuf[b], sem_w[b]).wait()

    def scatter_start(k, b):
        pltpu.async_copy(rows[b], acc.at[dst[b]], sem_s[b], add=True)

    def scatter_wait(k, b):
        pltpu.make_async_copy(rows[b], acc.at[dst[b]], sem_s[b]).wait()

    src_start(0, 0)
    src_start(1, 1)
    src_wait(0, 0)
    gather_start(0, 0)
    dw_start(0, 0)

    def group_body(g, _):
        for b in range(NBUF):
            k = g * NBUF + b
            pb = (b + 1) % NBUF
            nb = (b + 2) % NBUF

            @pl.when(k >= 2)
            def _():
                scatter_wait(k - 2, pb)

            @pl.when(k + 1 < cpw)
            def _():
                src_wait(k + 1, pb)
                gather_start(k + 1, pb)
                dw_start(k + 1, pb)

            @pl.when(k + 2 < cpw)
            def _():
                src_start(k + 2, nb)

            gather_wait(k, b)
            dw_wait(k, b)

            def mul_group(gg, _):
                w16 = wbuf[b][pl.ds(gg * 16, 16)]
                for l in range(16):
                    wv = jnp.full((16,), w16[l], jnp.float32)
                    i = gg * 16 + l
                    for j in range(D // 16):
                        sl = pl.ds(j * 16, 16)
                        rows[b][i, sl] = rows[b][i, sl] * wv
                return 0

            lax.fori_loop(0, CHUNK // 16, mul_group, 0)
            scatter_start(k, b)
        return 0

    lax.fori_loop(0, cpw // NBUF, group_body, 0)

    scatter_wait(cpw - 2, (cpw - 2) % NBUF)
    scatter_wait(cpw - 1, (cpw - 1) % NBUF)
    plsc.subcore_barrier()

    pltpu.sync_copy(acc.at[pl.ds(row0, ROWS_PER_TILE)],
                    out_hbm.at[cid, pl.ds(row0, ROWS_PER_TILE)])


def _scatter_gather(h, src_p, dst_p, w_p):
    mesh = plsc.VectorSubcoreMesh(core_axis_name="c", subcore_axis_name="s")
    run = pl.kernel(
        _sc_body,
        mesh=mesh,
        out_type=jax.ShapeDtypeStruct((NUM_CORES, N_PAD, D), jnp.float32),
        scratch_types=(
            [pltpu.VMEM((CHUNK, D), jnp.float32)] * NBUF
            + [pltpu.VMEM((CHUNK,), jnp.int32)] * NBUF
            + [pltpu.VMEM((CHUNK,), jnp.int32)] * NBUF
            + [pltpu.VMEM((CHUNK,), jnp.float32)] * NBUF
            + [pltpu.VMEM_SHARED((N_PAD, D), jnp.float32)]
            + [pltpu.SemaphoreType.DMA] * (5 * NBUF)
        ),
    )
    return run(h, src_p, dst_p, w_p)


@jax.jit
def kernel(x, edge_index, w, W, b):
    h = _linear(x, W.T, b[None, :])

    pad = E_PAD - E_EDGES
    pad_idx = (jnp.arange(pad, dtype=jnp.int32) * 37) % N_NODES
    edge_pad = jnp.concatenate(
        [edge_index, jnp.stack([pad_idx, pad_idx])], axis=1)
    w_pad = jnp.concatenate([w, jnp.zeros((pad,), jnp.float32)])

    partials = _scatter_gather(h, edge_pad[0], edge_pad[1], w_pad)
    return _combine(partials)
